# baseline (speedup 1.0000x reference)
import functools

import jax
import jax.numpy as jnp
from jax.experimental import pallas as pl

N_IN = 128
H1 = 128
H2 = 128
N_OUT = 64
TB = 8192


def _ffn_body(x_ref, w1_ref, b1_ref, m1_ref, w2_ref, b2_ref, m2_ref,
              w3_ref, b3_ref, m3_ref, o_ref):
    x = x_ref[...].astype(jnp.bfloat16)
    w1 = (w1_ref[...] * m1_ref[...]).astype(jnp.bfloat16)
    b1 = b1_ref[...].astype(jnp.bfloat16)
    h1 = jnp.maximum(
        jax.lax.dot_general(x, w1, (((1,), (1,)), ((), ())),
                            preferred_element_type=jnp.float32
                            ).astype(jnp.bfloat16) + b1,
        jnp.bfloat16(0.0))
    w2 = (w2_ref[...] * m2_ref[...]).astype(jnp.bfloat16)
    b2 = b2_ref[...].astype(jnp.bfloat16)
    h2 = jnp.maximum(
        jax.lax.dot_general(h1, w2, (((1,), (1,)), ((), ())),
                            preferred_element_type=jnp.float32
                            ).astype(jnp.bfloat16) + b2,
        jnp.bfloat16(0.0))
    w3 = (w3_ref[...] * m3_ref[...]).astype(jnp.bfloat16)
    o_ref[...] = jnp.maximum(
        jax.lax.dot_general(h2, w3, (((1,), (1,)), ((), ())),
                            preferred_element_type=jnp.float32)
        + b3_ref[...], 0.0)


@functools.partial(jax.jit, static_argnames=("interpret",))
def kernel(inputs, W1, b1, M1, W2, b2, M2, W3, b3, M3, interpret=False):
    B = inputs.shape[0]
    grid = (B // TB,)
    full = lambda i: (0, 0)
    wspec = lambda r, c: pl.BlockSpec((r, c), full)
    return pl.pallas_call(
        _ffn_body,
        grid=grid,
        in_specs=[
            pl.BlockSpec((TB, N_IN), lambda i: (i, 0)),
            wspec(H1, N_IN), wspec(1, H1), wspec(H1, N_IN),
            wspec(H2, H1), wspec(1, H2), wspec(H2, H1),
            wspec(N_OUT, H2), wspec(1, N_OUT), wspec(N_OUT, H2),
        ],
        out_specs=pl.BlockSpec((TB, N_OUT), lambda i: (i, 0)),
        out_shape=jax.ShapeDtypeStruct((B, N_OUT), jnp.float32),
        interpret=interpret,
    )(inputs, W1, b1.reshape(1, H1), M1,
      W2, b2.reshape(1, H2), M2,
      W3, b3.reshape(1, N_OUT), M3)


# speedup vs baseline: 1.0053x; 1.0053x over previous
import functools

import jax
import jax.numpy as jnp
from jax.experimental import pallas as pl
from jax.experimental.pallas import tpu as pltpu

N_IN = 128
H1 = 128
H2 = 128
N_OUT = 64
TB = 8192


def _ffn_body(x_ref, w1_ref, b1_ref, m1_ref, w2_ref, b2_ref, m2_ref,
              w3_ref, b3_ref, m3_ref, o_ref):
    x = x_ref[...].astype(jnp.bfloat16)
    w1 = (w1_ref[...] * m1_ref[...]).astype(jnp.bfloat16)
    b1 = b1_ref[...].astype(jnp.bfloat16)
    h1 = jnp.maximum(
        jax.lax.dot_general(x, w1, (((1,), (1,)), ((), ())),
                            preferred_element_type=jnp.float32
                            ).astype(jnp.bfloat16) + b1,
        jnp.bfloat16(0.0))
    w2 = (w2_ref[...] * m2_ref[...]).astype(jnp.bfloat16)
    b2 = b2_ref[...].astype(jnp.bfloat16)
    h2 = jnp.maximum(
        jax.lax.dot_general(h1, w2, (((1,), (1,)), ((), ())),
                            preferred_element_type=jnp.float32
                            ).astype(jnp.bfloat16) + b2,
        jnp.bfloat16(0.0))
    w3 = (w3_ref[...] * m3_ref[...]).astype(jnp.bfloat16)
    o_ref[...] = jnp.maximum(
        jax.lax.dot_general(h2, w3, (((1,), (1,)), ((), ())),
                            preferred_element_type=jnp.float32)
        + b3_ref[...], 0.0)


@functools.partial(jax.jit, static_argnames=("interpret",))
def kernel(inputs, W1, b1, M1, W2, b2, M2, W3, b3, M3, interpret=False):
    B = inputs.shape[0]
    grid = (B // TB,)
    full = lambda i: (0, 0)
    wspec = lambda r, c: pl.BlockSpec((r, c), full)
    return pl.pallas_call(
        _ffn_body,
        grid=grid,
        in_specs=[
            pl.BlockSpec((TB, N_IN), lambda i: (i, 0)),
            wspec(H1, N_IN), wspec(1, H1), wspec(H1, N_IN),
            wspec(H2, H1), wspec(1, H2), wspec(H2, H1),
            wspec(N_OUT, H2), wspec(1, N_OUT), wspec(N_OUT, H2),
        ],
        out_specs=pl.BlockSpec((TB, N_OUT), lambda i: (i, 0)),
        out_shape=jax.ShapeDtypeStruct((B, N_OUT), jnp.float32),
        compiler_params=pltpu.CompilerParams(
            dimension_semantics=("parallel",)),
        interpret=interpret,
    )(inputs, W1, b1.reshape(1, H1), M1,
      W2, b2.reshape(1, H2), M2,
      W3, b3.reshape(1, N_OUT), M3)
